# Initial kernel scaffold; baseline (speedup 1.0000x reference)
import jax
import jax.numpy as jnp
from jax.experimental import pallas as pl


def kernel(x, edge_index, batch, W1_root, W1_rel, b1, W2_root, W2_rel, b2):
    raise NotImplementedError("write your pallas kernel here")


# speedup vs baseline: 8.0243x; 8.0243x over previous
import functools

import jax
import jax.numpy as jnp
from jax import lax
from jax.experimental import pallas as pl
from jax.experimental.pallas import tpu as pltpu
from jax.experimental.pallas import tpu_sc as plsc

N = 10000
E = 320000
D = 128
G = 8

NC = 2
NS = 16
NW = NC * NS
C = 125
CHUNKS = E // C
CPW = CHUNKS // NW
SUP = 8
NSUP = CPW // SUP
RPT = N // NS
ZR = 125

_mesh = plsc.VectorSubcoreMesh(core_axis_name="c", subcore_axis_name="s")


@functools.partial(
    pl.kernel,
    mesh=_mesh,
    out_type=jax.ShapeDtypeStruct((NC, N, D), jnp.float32),
    scratch_types=[
        pltpu.VMEM((SUP, C), jnp.int32),
        pltpu.VMEM((SUP, C), jnp.int32),
        pltpu.VMEM((2, C, D), jnp.float32),
        pltpu.VMEM_SHARED((N, D), jnp.float32),
        pltpu.SemaphoreType.DMA,
        pltpu.SemaphoreType.DMA,
    ],
)
def _segsum_sc(y_hbm, src_hbm, dst_hbm, out_hbm,
               sidx, didx, rows, acc, sem0, sem1):
    c = lax.axis_index("c")
    s = lax.axis_index("s")
    wid = s * NC + c

    def _fill_zero(i, carry):
        r = i // 8
        q = (i % 8) * 16
        rows[0, r, pl.ds(q, 16)] = jnp.zeros((16,), jnp.float32)
        return carry

    lax.fori_loop(0, ZR * 8, _fill_zero, 0)
    for k in range(RPT // ZR):
        pltpu.sync_copy(rows.at[0], acc.at[pl.ds(s * RPT + k * ZR, ZR)])
    plsc.subcore_barrier()

    sems = [sem0, sem1]

    def _superchunk(g, carry):
        row0 = wid * CPW + g * SUP
        pltpu.sync_copy(src_hbm.at[pl.ds(row0, SUP)], sidx)
        pltpu.sync_copy(dst_hbm.at[pl.ds(row0, SUP)], didx)
        handles = [None, None]
        handles[0] = pltpu.async_copy(y_hbm.at[sidx.at[0]], rows.at[0],
                                      sems[0])
        for j in range(SUP):
            b = j % 2
            handles[b].wait()
            if j + 1 < SUP:
                nb = (j + 1) % 2
                handles[nb] = pltpu.async_copy(
                    y_hbm.at[sidx.at[j + 1]], rows.at[nb], sems[nb])
            pltpu.sync_copy(rows.at[b], acc.at[didx.at[j]], add=True)
        return carry

    lax.fori_loop(0, NSUP, _superchunk, 0)

    plsc.subcore_barrier()
    off = pl.multiple_of(s * 632, 8)

    @pl.when(s < NS - 1)
    def _drain_main():
        pltpu.sync_copy(acc.at[pl.ds(off, 632)],
                        out_hbm.at[c, pl.ds(off, 632)])

    @pl.when(s == NS - 1)
    def _drain_tail():
        pltpu.sync_copy(acc.at[pl.ds(off, 520)],
                        out_hbm.at[c, pl.ds(off, 520)])


def _segment_sum(y, src2, dst2):
    return _segsum_sc(y, src2, dst2)


BM = 1000
NB = N // BM


def _mm_body(x_ref, w_ref, o_ref):
    o_ref[...] = jnp.dot(x_ref[...], w_ref[...],
                         preferred_element_type=jnp.float32,
                         precision="highest")


def _matmul(x, w):
    return pl.pallas_call(
        _mm_body,
        grid=(NB,),
        in_specs=[pl.BlockSpec((BM, D), lambda i: (i, 0)),
                  pl.BlockSpec((D, D), lambda i: (0, 0))],
        out_specs=pl.BlockSpec((BM, D), lambda i: (i, 0)),
        out_shape=jax.ShapeDtypeStruct((N, D), jnp.float32),
    )(x, w)


def _fuse1_body(x_ref, m0_ref, m1_ref, wr_ref, b_ref, wn2_ref,
                h_ref, y_ref):
    m = m0_ref[...] + m1_ref[...]
    h = jnp.dot(x_ref[...], wr_ref[...],
                preferred_element_type=jnp.float32, precision="highest")
    h = jnp.maximum(h + m + b_ref[...], 0.0)
    h_ref[...] = h
    y_ref[...] = jnp.dot(h, wn2_ref[...],
                         preferred_element_type=jnp.float32,
                         precision="highest")


def _fuse1(x, m0, m1, w_root, b, w_rel2):
    return pl.pallas_call(
        _fuse1_body,
        grid=(NB,),
        in_specs=[pl.BlockSpec((BM, D), lambda i: (i, 0)),
                  pl.BlockSpec((BM, D), lambda i: (i, 0)),
                  pl.BlockSpec((BM, D), lambda i: (i, 0)),
                  pl.BlockSpec((D, D), lambda i: (0, 0)),
                  pl.BlockSpec((1, D), lambda i: (0, 0)),
                  pl.BlockSpec((D, D), lambda i: (0, 0))],
        out_specs=[pl.BlockSpec((BM, D), lambda i: (i, 0)),
                   pl.BlockSpec((BM, D), lambda i: (i, 0))],
        out_shape=[jax.ShapeDtypeStruct((N, D), jnp.float32),
                   jax.ShapeDtypeStruct((N, D), jnp.float32)],
    )(x, m0, m1, w_root, b, w_rel2)


def _fuse2_body(h_ref, m0_ref, m1_ref, wr_ref, b_ref, bat_ref, o_ref):
    i = pl.program_id(0)
    h2 = jnp.dot(h_ref[...], wr_ref[...],
                 preferred_element_type=jnp.float32, precision="highest")
    h2 = h2 + m0_ref[...] + m1_ref[...] + b_ref[...]
    bat = bat_ref[...]
    neg = jnp.float32(-jnp.inf)
    parts = []
    for g in range(G):
        mg = jnp.where(bat == g, h2, neg)
        parts.append(jnp.max(mg, axis=0, keepdims=True))
    blk = jnp.concatenate(parts, axis=0)

    @pl.when(i == 0)
    def _():
        o_ref[...] = jnp.full((G, D), neg, jnp.float32)

    o_ref[...] = jnp.maximum(o_ref[...], blk)


def _fuse2(h1, m0, m1, w_root, b, bat2d):
    return pl.pallas_call(
        _fuse2_body,
        grid=(NB,),
        in_specs=[pl.BlockSpec((BM, D), lambda i: (i, 0)),
                  pl.BlockSpec((BM, D), lambda i: (i, 0)),
                  pl.BlockSpec((BM, D), lambda i: (i, 0)),
                  pl.BlockSpec((D, D), lambda i: (0, 0)),
                  pl.BlockSpec((1, D), lambda i: (0, 0)),
                  pl.BlockSpec((BM, 1), lambda i: (i, 0))],
        out_specs=pl.BlockSpec((G, D), lambda i: (0, 0)),
        out_shape=jax.ShapeDtypeStruct((G, D), jnp.float32),
    )(h1, m0, m1, w_root, b, bat2d)


def kernel(x, edge_index, batch, W1_root, W1_rel, b1, W2_root, W2_rel, b2):
    src2 = edge_index[0].reshape(CHUNKS, C)
    dst2 = edge_index[1].reshape(CHUNKS, C)
    bat2d = batch.reshape(N, 1)
    b1r = b1.reshape(1, D)
    b2r = b2.reshape(1, D)

    y1 = _matmul(x, W1_rel)
    m1p = _segment_sum(y1, src2, dst2)
    h1, y2 = _fuse1(x, m1p[0], m1p[1], W1_root, b1r, W2_rel)
    m2p = _segment_sum(y2, src2, dst2)
    out = _fuse2(h1, m2p[0], m2p[1], W2_root, b2r, bat2d)
    return out
